# Initial kernel scaffold; baseline (speedup 1.0000x reference)
import jax
import jax.numpy as jnp
from jax.experimental import pallas as pl


def kernel(x, lut):
    raise NotImplementedError("write your pallas kernel here")


# speedup vs baseline: 6.2767x; 6.2767x over previous
import functools
import math

import jax
import jax.numpy as jnp
from jax.experimental import pallas as pl
from jax.experimental.pallas import tpu as pltpu
from jax.experimental.pallas import tpu_sc as plsc

D_MODEL = 128
SCALE = math.sqrt(D_MODEL)
WINDOW = 128


def _scale_lut(lut):
    v, d = lut.shape
    blk = 1000

    def body(l_ref, o_ref):
        o_ref[...] = l_ref[...] * SCALE

    return pl.pallas_call(
        body,
        out_shape=jax.ShapeDtypeStruct((v, d), jnp.float32),
        grid=(v // blk,),
        in_specs=[pl.BlockSpec((blk, d), lambda i: (i, 0))],
        out_specs=pl.BlockSpec((blk, d), lambda i: (i, 0)),
    )(lut)


def _sc_gather(scaled_lut, idx):
    b = idx.shape[0]
    idx2 = idx.reshape(1, b)
    mesh = plsc.VectorSubcoreMesh(core_axis_name="c", subcore_axis_name="s")

    @functools.partial(
        pl.kernel,
        out_type=jax.ShapeDtypeStruct((b, D_MODEL), jnp.float32),
        mesh=mesh,
    )
    def k(lut_hbm, i_hbm, o_hbm):
        def body(i_vmem, o_vmem):
            pltpu.sync_copy(lut_hbm.at[i_vmem.at[0]], o_vmem)

        pltpu.emit_pipeline(
            body,
            grid=(b // WINDOW,),
            in_specs=[pl.BlockSpec((1, WINDOW), lambda i: (0, i))],
            out_specs=[pl.BlockSpec((WINDOW, D_MODEL), lambda i: (i, 0))],
            core_axis_name=("c", "s"),
            dimension_semantics=(pltpu.PARALLEL,),
        )(i_hbm, o_hbm)

    return k(scaled_lut, idx2)


def kernel(x, lut):
    rows, cols = x.shape
    idx = x.reshape(-1).astype(jnp.int32)
    scaled = _scale_lut(lut)
    out = _sc_gather(scaled, idx)
    return out.reshape(rows, cols, D_MODEL)


# speedup vs baseline: 7.5752x; 1.2069x over previous
import functools
import math

import jax
import jax.numpy as jnp
from jax.experimental import pallas as pl
from jax.experimental.pallas import tpu as pltpu
from jax.experimental.pallas import tpu_sc as plsc

D_MODEL = 128
SCALE = math.sqrt(D_MODEL)
WINDOW = 128


def _scale_lut(lut):
    v, d = lut.shape
    blk = 1000

    def body(l_ref, o_ref):
        o_ref[...] = l_ref[...] * SCALE

    return pl.pallas_call(
        body,
        out_shape=jax.ShapeDtypeStruct((v, d), jnp.float32),
        grid=(v // blk,),
        in_specs=[pl.BlockSpec((blk, d), lambda i: (i, 0))],
        out_specs=pl.BlockSpec((blk, d), lambda i: (i, 0)),
    )(lut)


NC = 2
NS = 16
NW = NC * NS


RING = 4


def _sc_gather(scaled_lut, idx):
    from jax import lax

    b = idx.shape[0]
    b_per_w = b // NW
    n_chunks = b_per_w // WINDOW
    assert n_chunks % RING == 0 and n_chunks > RING
    mesh = plsc.VectorSubcoreMesh(core_axis_name="c", subcore_axis_name="s")

    scratch = (
        [pltpu.VMEM((WINDOW,), jnp.int32) for _ in range(RING)]
        + [pltpu.VMEM((WINDOW, D_MODEL), jnp.float32) for _ in range(RING)]
        + [pltpu.SemaphoreType.DMA for _ in range(2 * RING)]
    )

    @functools.partial(
        pl.kernel,
        out_type=jax.ShapeDtypeStruct((b, D_MODEL), jnp.float32),
        mesh=mesh,
        scratch_types=scratch,
    )
    def k(lut_hbm, i_hbm, o_hbm, *scr):
        idx_v = scr[:RING]
        buf = scr[RING : 2 * RING]
        gsem = scr[2 * RING : 3 * RING]
        osem = scr[3 * RING :]

        wid = lax.axis_index("c") * NS + lax.axis_index("s")
        base = wid * b_per_w

        def gather(j, bslot):
            return pltpu.make_async_copy(
                lut_hbm.at[idx_v[bslot]], buf[bslot], gsem[bslot]
            )

        def out_copy(j, bslot):
            return pltpu.make_async_copy(
                buf[bslot], o_hbm.at[pl.ds(base + j * WINDOW, WINDOW)], osem[bslot]
            )

        def load_idx(j, bslot):
            pltpu.sync_copy(i_hbm.at[pl.ds(base + j * WINDOW, WINDOW)], idx_v[bslot])

        for s in range(RING):
            load_idx(s, s)
            gather(s, s).start()

        @pl.loop(0, n_chunks - RING, step=RING)
        def _(g):
            for s in range(RING):
                gather(g + s, s).wait()
                out_copy(g + s, s).start()
            for s in range(RING):
                out_copy(g + s, s).wait()
                load_idx(g + RING + s, s)
                gather(g + RING + s, s).start()

        for s in range(RING):
            g = n_chunks - RING + s
            gather(g, s).wait()
            out_copy(g, s).start()
        for s in range(RING):
            out_copy(n_chunks - RING + s, s).wait()

    return k(scaled_lut, idx)


def kernel(x, lut):
    rows, cols = x.shape
    idx = x.reshape(-1).astype(jnp.int32)
    scaled = _scale_lut(lut)
    out = _sc_gather(scaled, idx)
    return out.reshape(rows, cols, D_MODEL)


# speedup vs baseline: 7.9622x; 1.0511x over previous
import functools
import math

import jax
import jax.numpy as jnp
from jax.experimental import pallas as pl
from jax.experimental.pallas import tpu as pltpu
from jax.experimental.pallas import tpu_sc as plsc

D_MODEL = 128
SCALE = math.sqrt(D_MODEL)
WINDOW = 128


def _scale_lut(lut):
    v, d = lut.shape
    blk = 2000

    def body(l_ref, o_ref):
        o_ref[...] = l_ref[...] * SCALE

    return pl.pallas_call(
        body,
        out_shape=jax.ShapeDtypeStruct((v, d), jnp.float32),
        grid=(v // blk,),
        in_specs=[pl.BlockSpec((blk, d), lambda i: (i, 0))],
        out_specs=pl.BlockSpec((blk, d), lambda i: (i, 0)),
    )(lut)


NC = 2
NS = 16
NW = NC * NS


RING = 5


def _sc_gather(scaled_lut, idx):
    from jax import lax

    b = idx.shape[0]
    b_per_w = b // NW
    n_chunks = b_per_w // WINDOW
    assert n_chunks % RING == 0 and n_chunks > RING
    mesh = plsc.VectorSubcoreMesh(core_axis_name="c", subcore_axis_name="s")

    scratch = (
        [pltpu.VMEM((WINDOW,), jnp.int32) for _ in range(RING)]
        + [pltpu.VMEM((WINDOW, D_MODEL), jnp.float32) for _ in range(RING)]
        + [pltpu.SemaphoreType.DMA for _ in range(2 * RING)]
    )

    @functools.partial(
        pl.kernel,
        out_type=jax.ShapeDtypeStruct((b, D_MODEL), jnp.float32),
        mesh=mesh,
        scratch_types=scratch,
    )
    def k(lut_hbm, i_hbm, o_hbm, *scr):
        idx_v = scr[:RING]
        buf = scr[RING : 2 * RING]
        gsem = scr[2 * RING : 3 * RING]
        osem = scr[3 * RING :]

        wid = lax.axis_index("c") * NS + lax.axis_index("s")
        base = wid * b_per_w

        def gather(j, bslot):
            return pltpu.make_async_copy(
                lut_hbm.at[idx_v[bslot]], buf[bslot], gsem[bslot]
            )

        def out_copy(j, bslot):
            return pltpu.make_async_copy(
                buf[bslot], o_hbm.at[pl.ds(base + j * WINDOW, WINDOW)], osem[bslot]
            )

        def load_idx(j, bslot):
            pltpu.sync_copy(i_hbm.at[pl.ds(base + j * WINDOW, WINDOW)], idx_v[bslot])

        for s in range(RING):
            load_idx(s, s)
            gather(s, s).start()

        @pl.loop(0, n_chunks - RING, step=RING)
        def _(g):
            for s in range(RING):
                gather(g + s, s).wait()
                out_copy(g + s, s).start()
            for s in range(RING):
                out_copy(g + s, s).wait()
                load_idx(g + RING + s, s)
                gather(g + RING + s, s).start()

        for s in range(RING):
            g = n_chunks - RING + s
            gather(g, s).wait()
            out_copy(g, s).start()
        for s in range(RING):
            out_copy(n_chunks - RING + s, s).wait()

    return k(scaled_lut, idx)


def kernel(x, lut):
    rows, cols = x.shape
    idx = x.reshape(-1).astype(jnp.int32)
    scaled = _scale_lut(lut)
    out = _sc_gather(scaled, idx)
    return out.reshape(rows, cols, D_MODEL)
